# baseline (speedup 1.0000x reference)
import jax
import jax.numpy as jnp
import numpy as np
from jax import lax
from jax.experimental import pallas as pl
from jax.experimental.pallas import tpu as pltpu
from jax.experimental.pallas import tpu_sc as plsc

_B = 64
_T = 160000
_N = 2 * _B

_NC, _NS = 2, 16
_NW = _NC * _NS
_CH = 3200
_NCH = _T // _CH
_NCHT = _N * _NCH
_CPW = _NCHT // _NW
_G = 8
_STEPS = _CPW // _G

def _threefry2x32(k0, k1, x0, x1):
    rotations = [[13, 15, 26, 6], [17, 29, 16, 24]]

    def rol(x, d):
        return ((x << np.uint32(d)) | (x >> np.uint32(32 - d))).astype(np.uint32)

    ks = [np.uint32(k0), np.uint32(k1),
          np.uint32(np.uint32(k0) ^ np.uint32(k1) ^ np.uint32(0x1BD11BDA))]
    x = [x0.astype(np.uint32) + ks[0], x1.astype(np.uint32) + ks[1]]
    for i in range(5):
        for r in rotations[i % 2]:
            x[0] = (x[0] + x[1]).astype(np.uint32)
            x[1] = rol(x[1], r)
            x[1] = x[1] ^ x[0]
        x[0] = (x[0] + ks[(i + 1) % 3]).astype(np.uint32)
        x[1] = (x[1] + ks[(i + 2) % 3] + np.uint32(i + 1)).astype(np.uint32)
    return x


def _gather_idx() -> np.ndarray:
    old = np.seterr(over="ignore")
    r0, r1 = _threefry2x32(0, 42, np.zeros(_B, np.uint32), np.arange(_B, dtype=np.uint32))
    np.seterr(**old)
    bits = r0 ^ r1
    u = ((bits >> np.uint32(9)) | np.uint32(0x3F800000)).view(np.float32) - np.float32(1.0)
    perm = np.argsort(u, kind="stable")
    return np.concatenate([perm, _B + np.arange(_B)]).astype(np.int32)


def _chunk_idx() -> np.ndarray:
    g = _gather_idx()
    return (g[:, None] * _NCH + np.arange(_NCH)[None, :]).reshape(-1).astype(np.int32)


def _sc_body(cidx_hbm, src_hbm, out_hbm, idx_v, bufs, in_sems, out_sems):
    wid = lax.axis_index("s") * _NC + lax.axis_index("c")
    base = wid * _CPW
    pltpu.sync_copy(cidx_hbm.at[pl.ds(base, _CPW)], idx_v)

    def in_cp(g, b):
        return pltpu.make_async_copy(
            src_hbm.at[idx_v.at[pl.ds(g * _G, _G)]], bufs.at[b], in_sems.at[b])

    def out_cp(g, b):
        return pltpu.make_async_copy(
            bufs.at[b], out_hbm.at[pl.ds(base + g * _G, _G)], out_sems.at[b])

    in_cp(0, 0).start()
    in_cp(1, 1).start()

    def step(h, c):
        for b in range(2):
            g = h * 2 + b
            in_cp(g, b).wait()
            out_cp(g, b).start()

            @pl.when(g + 2 < _STEPS)
            def _refill():
                out_cp(g, b).wait()
                in_cp(g + 2, b).start()

        return c

    lax.fori_loop(0, _STEPS // 2, step, 0)
    if _STEPS % 2:
        g = _STEPS - 1
        in_cp(g, 0).wait()
        out_cp(g, 0).start()
        out_cp(_STEPS - 2, 1).wait()
        out_cp(_STEPS - 1, 0).wait()
    else:
        out_cp(_STEPS - 2, 0).wait()
        out_cp(_STEPS - 1, 1).wait()


def kernel(sources):
    cidx = jnp.asarray(_chunk_idx())
    src = sources.reshape(_NCHT, _CH)
    mesh = plsc.VectorSubcoreMesh(core_axis_name="c", subcore_axis_name="s")
    out = pl.kernel(
        _sc_body,
        out_type=jax.ShapeDtypeStruct((_NCHT, _CH), jnp.float32),
        mesh=mesh,
        scratch_types=[
            pltpu.VMEM((_CPW,), jnp.int32),
            pltpu.VMEM((2, _G, _CH), jnp.float32),
            pltpu.SemaphoreType.DMA((2,)),
            pltpu.SemaphoreType.DMA((2,)),
        ],
    )(cidx, src)
    return out.reshape(2, _B, 1, _T)
